# baseline (speedup 1.0000x reference)
import functools

import jax
import jax.numpy as jnp
from jax import lax
from jax.experimental import pallas as pl
from jax.experimental.pallas import tpu as pltpu
from jax.experimental.pallas import tpu_sc as plsc

VOCAB = 100000
DIM = 128
B = 1024
L = 50

NC = 2
NS = 16
NW = NC * NS
RPW = B // NW
NLANE = DIM // 16


def _sc_pool_body(idx_hbm, table_hbm, out_hbm, idx_v, rows_v, out_v, sem):
    wid = lax.axis_index("s") * NC + lax.axis_index("c")
    pltpu.sync_copy(idx_hbm.at[wid], idx_v)

    def row_body(r, carry):
        pltpu.async_copy(table_hbm.at[idx_v.at[r]], rows_v, sem).wait()

        def lane_acc(l, accs):
            return tuple(accs[d] + rows_v[l, pl.ds(d * 16, 16)]
                         for d in range(NLANE))

        accs = tuple(rows_v[0, pl.ds(d * 16, 16)] for d in range(NLANE))
        accs = lax.fori_loop(1, L, lane_acc, accs)
        for d in range(NLANE):
            out_v[r, pl.ds(d * 16, 16)] = accs[d] * (1.0 / L)
        return carry

    lax.fori_loop(0, RPW, row_body, 0)
    pltpu.sync_copy(out_v, out_hbm.at[pl.ds(wid * RPW, RPW)])


_sc_pool = functools.partial(
    pl.kernel,
    out_type=jax.ShapeDtypeStruct((B, DIM), jnp.float32),
    mesh=plsc.VectorSubcoreMesh(core_axis_name="c", subcore_axis_name="s"),
    scratch_types=[
        pltpu.VMEM((RPW, L), jnp.int32),
        pltpu.VMEM((L, DIM), jnp.float32),
        pltpu.VMEM((RPW, DIM), jnp.float32),
        pltpu.SemaphoreType.DMA,
    ],
)(_sc_pool_body)


VBLK = 1024


def _mm_body(p_ref, w_ref, b_ref, o_ref):
    acc = lax.dot_general(p_ref[...], w_ref[...],
                          (((1,), (0,)), ((), ())),
                          preferred_element_type=jnp.float32)
    o_ref[...] = acc + b_ref[...]


def _decoder(pooled, Wt, b2d):
    grid = pl.cdiv(VOCAB, VBLK)
    return pl.pallas_call(
        _mm_body,
        grid=(grid,),
        in_specs=[
            pl.BlockSpec((B, DIM), lambda i: (0, 0)),
            pl.BlockSpec((DIM, VBLK), lambda i: (0, i)),
            pl.BlockSpec((1, VBLK), lambda i: (0, i)),
        ],
        out_specs=pl.BlockSpec((B, VBLK), lambda i: (0, i)),
        out_shape=jax.ShapeDtypeStruct((B, VOCAB), jnp.float32),
    )(pooled, Wt, b2d)


def kernel(contextTsr, emb_table, W, b):
    idx = contextTsr.reshape(NW, RPW, L)
    pooled = _sc_pool(idx, emb_table)
    return _decoder(pooled, W.T, b.reshape(1, VOCAB))


# speedup vs baseline: 1.0352x; 1.0352x over previous
import functools

import jax
import jax.numpy as jnp
from jax import lax
from jax.experimental import pallas as pl
from jax.experimental.pallas import tpu as pltpu
from jax.experimental.pallas import tpu_sc as plsc

VOCAB = 100000
DIM = 128
B = 1024
L = 50

NC = 2
NS = 16
NW = NC * NS
RPW = B // NW
NLANE = DIM // 16


def _sc_pool_body(idx_hbm, table_hbm, out_hbm, idx_v, rows_v, out_v, sem):
    wid = lax.axis_index("s") * NC + lax.axis_index("c")
    pltpu.sync_copy(idx_hbm.at[wid], idx_v)

    def row_body(r, carry):
        pltpu.async_copy(table_hbm.at[idx_v.at[r]], rows_v, sem).wait()

        def lane_acc(l, accs):
            return tuple(accs[d] + rows_v[l, pl.ds(d * 16, 16)]
                         for d in range(NLANE))

        accs = tuple(rows_v[0, pl.ds(d * 16, 16)] for d in range(NLANE))
        accs = lax.fori_loop(1, L, lane_acc, accs)
        for d in range(NLANE):
            out_v[r, pl.ds(d * 16, 16)] = accs[d] * (1.0 / L)
        return carry

    lax.fori_loop(0, RPW, row_body, 0)
    pltpu.sync_copy(out_v, out_hbm.at[pl.ds(wid * RPW, RPW)])


_sc_pool = functools.partial(
    pl.kernel,
    out_type=jax.ShapeDtypeStruct((B, DIM), jnp.float32),
    mesh=plsc.VectorSubcoreMesh(core_axis_name="c", subcore_axis_name="s"),
    scratch_types=[
        pltpu.VMEM((RPW, L), jnp.int32),
        pltpu.VMEM((L, DIM), jnp.float32),
        pltpu.VMEM((RPW, DIM), jnp.float32),
        pltpu.SemaphoreType.DMA,
    ],
)(_sc_pool_body)


VBLK = 2048


def _mm_body(p_ref, w_ref, b_ref, o_ref):
    acc = lax.dot_general(p_ref[...], w_ref[...],
                          (((1,), (0,)), ((), ())),
                          preferred_element_type=jnp.float32)
    o_ref[...] = acc + b_ref[...]


def _decoder(pooled, Wt, b2d):
    grid = pl.cdiv(VOCAB, VBLK)
    return pl.pallas_call(
        _mm_body,
        grid=(grid,),
        in_specs=[
            pl.BlockSpec((B, DIM), lambda i: (0, 0)),
            pl.BlockSpec((DIM, VBLK), lambda i: (0, i)),
            pl.BlockSpec((1, VBLK), lambda i: (0, i)),
        ],
        out_specs=pl.BlockSpec((B, VBLK), lambda i: (0, i)),
        out_shape=jax.ShapeDtypeStruct((B, VOCAB), jnp.float32),
    )(pooled, Wt, b2d)


def kernel(contextTsr, emb_table, W, b):
    idx = contextTsr.reshape(NW, RPW, L)
    pooled = _sc_pool(idx, emb_table)
    return _decoder(pooled, W.T, b.reshape(1, VOCAB))


# speedup vs baseline: 1.0980x; 1.0607x over previous
import functools

import jax
import jax.numpy as jnp
from jax import lax
from jax.experimental import pallas as pl
from jax.experimental.pallas import tpu as pltpu
from jax.experimental.pallas import tpu_sc as plsc

VOCAB = 100000
DIM = 128
B = 1024
L = 50

NC = 2
NS = 16
NW = NC * NS
RPW = B // NW
NLANE = DIM // 16


def _sc_pool_body(idx_hbm, table_hbm, out_hbm, idx_v, rows_v, out_v, sem):
    wid = lax.axis_index("s") * NC + lax.axis_index("c")
    pltpu.sync_copy(idx_hbm.at[wid], idx_v)

    def row_body(r, carry):
        pltpu.async_copy(table_hbm.at[idx_v.at[r]], rows_v, sem).wait()

        def lane_acc(l, accs):
            return tuple(accs[d] + rows_v[l, pl.ds(d * 16, 16)]
                         for d in range(NLANE))

        accs = tuple(rows_v[0, pl.ds(d * 16, 16)] for d in range(NLANE))
        accs = lax.fori_loop(1, L, lane_acc, accs)
        for d in range(NLANE):
            out_v[r, pl.ds(d * 16, 16)] = accs[d] * (1.0 / L)
        return carry

    lax.fori_loop(0, RPW, row_body, 0)
    pltpu.sync_copy(out_v, out_hbm.at[pl.ds(wid * RPW, RPW)])


_sc_pool = functools.partial(
    pl.kernel,
    out_type=jax.ShapeDtypeStruct((B, DIM), jnp.float32),
    mesh=plsc.VectorSubcoreMesh(core_axis_name="c", subcore_axis_name="s"),
    scratch_types=[
        pltpu.VMEM((RPW, L), jnp.int32),
        pltpu.VMEM((L, DIM), jnp.float32),
        pltpu.VMEM((RPW, DIM), jnp.float32),
        pltpu.SemaphoreType.DMA,
    ],
)(_sc_pool_body)


VBLK = 1024
NBLK = (VOCAB + VBLK - 1) // VBLK
TAIL = VOCAB - (NBLK - 1) * VBLK
NBUF = 6


def _mm_body(p_ref, b_ref, w_hbm, out_hbm, wbuf, obuf, wsem, osem):
    def w_copy(i, slot, sz):
        return pltpu.make_async_copy(
            w_hbm.at[pl.ds(i * VBLK, sz), :],
            wbuf.at[slot, pl.ds(0, sz), :],
            wsem.at[slot])

    def o_copy(i, slot, sz):
        return pltpu.make_async_copy(
            obuf.at[slot, :, pl.ds(0, sz)],
            out_hbm.at[:, pl.ds(i * VBLK, sz)],
            osem.at[slot])

    NFULL = NBLK - 1
    w_copy(0, 0, VBLK).start()
    w_copy(1, 1, VBLK).start()
    p = p_ref[...]

    def step(i, carry):
        slot = lax.rem(i, 2)
        ob = lax.rem(i, NBUF)
        w_copy(i, slot, VBLK).wait()
        acc = lax.dot_general(p, wbuf[slot],
                              (((1,), (1,)), ((), ())),
                              preferred_element_type=jnp.float32)
        acc = acc + b_ref[pl.ds(i, 1), :]

        @pl.when(i >= NBUF)
        def _():
            o_copy(i - NBUF, ob, VBLK).wait()

        obuf[ob] = acc
        o_copy(i, ob, VBLK).start()

        @pl.when(i + 2 < NFULL)
        def _():
            w_copy(i + 2, slot, VBLK).start()

        return carry

    lax.fori_loop(0, NFULL, step, 0)

    for j in range(NFULL - NBUF, NFULL):
        o_copy(j, j % NBUF, VBLK).wait()


def _tail_body(alias_ref, p_ref, w_ref, b_ref, o_ref):
    acc = lax.dot_general(p_ref[...], w_ref[...],
                          (((1,), (1,)), ((), ())),
                          preferred_element_type=jnp.float32)
    o_ref[...] = acc + b_ref[...]


def _decoder(pooled, W, b_pad):
    out = pl.pallas_call(
        _mm_body,
        in_specs=[
            pl.BlockSpec(memory_space=pltpu.VMEM),
            pl.BlockSpec(memory_space=pltpu.VMEM),
            pl.BlockSpec(memory_space=pl.ANY),
        ],
        out_specs=pl.BlockSpec(memory_space=pl.ANY),
        out_shape=jax.ShapeDtypeStruct((B, VOCAB), jnp.float32),
        scratch_shapes=[
            pltpu.VMEM((2, VBLK, DIM), jnp.float32),
            pltpu.VMEM((NBUF, B, VBLK), jnp.float32),
            pltpu.SemaphoreType.DMA((2,)),
            pltpu.SemaphoreType.DMA((NBUF,)),
        ],
    )(pooled, b_pad, W)
    return pl.pallas_call(
        _tail_body,
        grid=(1,),
        in_specs=[
            pl.BlockSpec(memory_space=pl.ANY),
            pl.BlockSpec((B, DIM), lambda i: (0, 0)),
            pl.BlockSpec((VBLK, DIM), lambda i: (NBLK - 1, 0)),
            pl.BlockSpec((1, VBLK), lambda i: (0, NBLK - 1)),
        ],
        out_specs=pl.BlockSpec((B, VBLK), lambda i: (0, NBLK - 1)),
        out_shape=jax.ShapeDtypeStruct((B, VOCAB), jnp.float32),
        input_output_aliases={0: 0},
    )(out, pooled, W, b_pad.reshape(1, NBLK * VBLK))


def kernel(contextTsr, emb_table, W, b):
    idx = contextTsr.reshape(NW, RPW, L)
    pooled = _sc_pool(idx, emb_table)
    b_pad = jnp.zeros((NBLK * VBLK,), jnp.float32).at[:VOCAB].set(b)
    return _decoder(pooled, W, b_pad.reshape(NBLK, VBLK))
